# baseline (speedup 1.0000x reference)
import functools

import jax
import jax.numpy as jnp
from jax import lax
from jax.experimental import pallas as pl
from jax.experimental.pallas import tpu as pltpu
from jax.experimental.pallas import tpu_sc as plsc

B, S, K, N = 4, 8192, 2048, 768
R, L = 64, 128
NW = 32


_GB = 4


def _sort_body(l0_ref, l1_ref, ranks_ref):
    b = pl.program_id(0)
    l0 = l0_ref[...]
    l1 = l1_ref[...]
    m = jnp.maximum(l0, l1)
    e0 = jnp.exp(l0 - m)
    e1 = jnp.exp(l1 - m)
    key = jnp.maximum(e0, e1) / (e0 + e1)

    ri = lax.broadcasted_iota(jnp.int32, (_GB, R, L), 1)
    li = lax.broadcasted_iota(jnp.int32, (_GB, R, L), 2)
    bi = lax.broadcasted_iota(jnp.int32, (_GB, R, L), 0)
    gi = li * R + ri
    tok = ri * L + li
    idx = tok + (b * _GB + bi) * S

    def partner(x, mj, sh, ax):
        size = (_GB, R, L)[ax]
        return jnp.where(mj, pltpu.roll(x, sh, ax),
                         pltpu.roll(x, size - sh, ax))

    k = 2
    while k <= S:
        mk = (gi & k) != 0
        j = k // 2
        while j >= 1:
            mj = (gi & j) != 0
            ax, sh = (2, j // R) if j >= R else (1, j)
            pk = partner(key, mj, sh, ax)
            pi = partner(idx, mj, sh, ax)
            before = (pk > key) | ((pk == key) & (pi < idx))
            take = before ^ mj ^ mk
            key = jnp.where(take, pk, key)
            idx = jnp.where(take, pi, idx)
            j //= 2
        k *= 2
    ranks_ref[...] = jnp.swapaxes(idx, 1, 2)


def _sort_call(l0, l1, interpret=False):
    return pl.pallas_call(
        _sort_body,
        grid=(B // _GB,),
        in_specs=[
            pl.BlockSpec((_GB, R, L), lambda b: (b, 0, 0)),
            pl.BlockSpec((_GB, R, L), lambda b: (b, 0, 0)),
        ],
        out_specs=pl.BlockSpec((_GB, L, R), lambda b: (b, 0, 0)),
        out_shape=jax.ShapeDtypeStruct((B, L, R), jnp.int32),
        interpret=interpret,
    )(l0, l1)


_FCH = 32
_NCH = 8
_NBF = 4


def _gather_body(feats_hbm, ranks2d_hbm, ranks8_hbm, l0_hbm, l1_hbm,
                 sf_hbm, p1_hbm, p0_hbm,
                 fidx, fbuf0, fbuf1, fbuf2, fbuf3, pidx, lbuf0, lbuf1,
                 stg0, stg1, gsem0, gsem1, gsem2, gsem3,
                 ssem0, ssem1, ssem2, ssem3):
    fbufs = (fbuf0, fbuf1, fbuf2, fbuf3)
    gsems = (gsem0, gsem1, gsem2, gsem3)
    ssems = (ssem0, ssem1, ssem2, ssem3)
    wid = lax.axis_index("s") * 2 + lax.axis_index("c")

    b = wid // 8
    row0 = pl.multiple_of(256 * b + 8 * (wid % 8), 8)
    pltpu.sync_copy(ranks2d_hbm.at[pl.ds(row0, _NCH)], fidx)

    def fire_gather(c):
        return pltpu.async_copy(feats_hbm.at[fidx.at[c]], fbufs[c % _NBF],
                                gsems[c % _NBF])

    def fire_store(c):
        out0 = pl.multiple_of(256 * wid + _FCH * c, _FCH)
        return pltpu.async_copy(fbufs[c % _NBF], sf_hbm.at[pl.ds(out0, _FCH)],
                                ssems[c % _NBF])

    gcp = [None] * _NBF
    scp = [None] * _NBF
    for c in range(_NBF):
        gcp[c] = fire_gather(c)
    for c in range(_NCH):
        if 1 <= c <= _NCH - _NBF:
            scp[(c - 1) % _NBF].wait()
            gcp[(c - 1) % _NBF] = fire_gather(c + _NBF - 1)
        gcp[c % _NBF].wait()
        scp[c % _NBF] = fire_store(c)
    for c in range(_NCH - _NBF, _NCH):
        scp[c % _NBF].wait()

    pltpu.sync_copy(ranks8_hbm.at[pl.ds(pl.multiple_of(8 * wid, 8), 8)], pidx)
    pltpu.sync_copy(l0_hbm.at[pl.ds(pl.multiple_of(b * 64, 64), 64)], lbuf0)
    pltpu.sync_copy(l1_hbm.at[pl.ds(pl.multiple_of(b * 64, 64), 64)], lbuf1)
    base_flat = b * S
    for v in range(64):
        g = pidx[v // 8, pl.ds((v % 8) * 16, 16)]
        e = g - base_flat
        er, ec = e >> 7, e & 127
        g0 = plsc.load_gather(lbuf0, [er, ec])
        g1 = plsc.load_gather(lbuf1, [er, ec])
        stg0[v // 8, pl.ds((v % 8) * 16, 16)] = g0
        stg1[v // 8, pl.ds((v % 8) * 16, 16)] = g1
    jj0 = 1024 * (wid % 8)

    @pl.when(jj0 < K)
    def _():
        crow = pl.multiple_of(jj0 // 128, 8)
        pltpu.sync_copy(stg0, p1_hbm.at[2 * b, pl.ds(crow, 8)])
        pltpu.sync_copy(stg1, p1_hbm.at[2 * b + 1, pl.ds(crow, 8)])

    @pl.when(jj0 >= K)
    def _():
        crow = pl.multiple_of((jj0 - K) // 128, 8)
        pltpu.sync_copy(stg0, p0_hbm.at[2 * b, pl.ds(crow, 8)])
        pltpu.sync_copy(stg1, p0_hbm.at[2 * b + 1, pl.ds(crow, 8)])


@functools.lru_cache(maxsize=None)
def _build_gather():
    return pl.kernel(
        _gather_body,
        out_type=(
            jax.ShapeDtypeStruct((B * K, N), jnp.float32),
            jax.ShapeDtypeStruct((2 * B, K // 128, 128), jnp.float32),
            jax.ShapeDtypeStruct((2 * B, (S - K) // 128, 128), jnp.float32),
        ),
        mesh=plsc.VectorSubcoreMesh(core_axis_name="c", subcore_axis_name="s"),
        compiler_params=pltpu.CompilerParams(needs_layout_passes=False),
        scratch_types=[
            pltpu.VMEM((_NCH, _FCH), jnp.int32),
            pltpu.VMEM((_FCH, N), jnp.float32),
            pltpu.VMEM((_FCH, N), jnp.float32),
            pltpu.VMEM((_FCH, N), jnp.float32),
            pltpu.VMEM((_FCH, N), jnp.float32),
            pltpu.VMEM((8, 128), jnp.int32),
            pltpu.VMEM((64, 128), jnp.float32),
            pltpu.VMEM((64, 128), jnp.float32),
            pltpu.VMEM((8, 128), jnp.float32),
            pltpu.VMEM((8, 128), jnp.float32),
            pltpu.SemaphoreType.DMA,
            pltpu.SemaphoreType.DMA,
            pltpu.SemaphoreType.DMA,
            pltpu.SemaphoreType.DMA,
            pltpu.SemaphoreType.DMA,
            pltpu.SemaphoreType.DMA,
            pltpu.SemaphoreType.DMA,
            pltpu.SemaphoreType.DMA,
        ],
    )


def _gather_call(feats2d, ranks2d, ranks8, l0p, l1p):
    return _build_gather()(feats2d, ranks2d, ranks8, l0p, l1p)


def kernel(feats, logit):
    l0 = logit[..., 0].reshape(B, R, L)
    l1 = logit[..., 1].reshape(B, R, L)
    ranks = _sort_call(l0, l1)
    ranks2d = ranks.reshape(B * S // _FCH, _FCH)
    ranks8 = ranks.reshape(B * S // 128, 128)
    feats2d = feats.reshape(B * S, N)
    l0p = l0.reshape(B * S // 128, 128)
    l1p = l1.reshape(B * S // 128, 128)
    sf2d, p1t, p0t = _gather_call(feats2d, ranks2d, ranks8, l0p, l1p)
    p1 = p1t.reshape(B, 2, K).swapaxes(1, 2)
    p0 = p0t.reshape(B, 2, S - K).swapaxes(1, 2)
    return (sf2d.reshape(B, K, N), p1, p0)


# speedup vs baseline: 1.0575x; 1.0575x over previous
import functools

import jax
import jax.numpy as jnp
from jax import lax
from jax.experimental import pallas as pl
from jax.experimental.pallas import tpu as pltpu
from jax.experimental.pallas import tpu_sc as plsc

B, S, K, N = 4, 8192, 2048, 768
R, L = 64, 128
NW = 32


_GB = 4


def _sort_body(l0_ref, l1_ref, ranks_ref, l0o_ref, l1o_ref):
    b = pl.program_id(0)
    l0 = l0_ref[...]
    l1 = l1_ref[...]
    l0o_ref[...] = l0
    l1o_ref[...] = l1
    m = jnp.maximum(l0, l1)
    e0 = jnp.exp(l0 - m)
    e1 = jnp.exp(l1 - m)
    key = jnp.maximum(e0, e1) / (e0 + e1)

    ri = lax.broadcasted_iota(jnp.int32, (_GB, R, L), 1)
    li = lax.broadcasted_iota(jnp.int32, (_GB, R, L), 2)
    bi = lax.broadcasted_iota(jnp.int32, (_GB, R, L), 0)
    gi = li * R + ri
    tok = ri * L + li
    idx = tok + (b * _GB + bi) * S

    def partner(x, mj, sh, ax):
        size = (_GB, R, L)[ax]
        return jnp.where(mj, pltpu.roll(x, sh, ax),
                         pltpu.roll(x, size - sh, ax))

    k = 2
    while k <= S:
        mk = (gi & k) != 0
        j = k // 2
        while j >= 1:
            mj = (gi & j) != 0
            ax, sh = (2, j // R) if j >= R else (1, j)
            pk = partner(key, mj, sh, ax)
            pi = partner(idx, mj, sh, ax)
            before = (pk > key) | ((pk == key) & (pi < idx))
            take = before ^ mj ^ mk
            key = jnp.where(take, pk, key)
            idx = jnp.where(take, pi, idx)
            j //= 2
        k *= 2
    ranks_ref[...] = jnp.swapaxes(idx, 1, 2)


def _sort_call(l0, l1, interpret=False):
    return pl.pallas_call(
        _sort_body,
        grid=(B // _GB,),
        in_specs=[
            pl.BlockSpec((_GB, R, L), lambda b: (b, 0, 0)),
            pl.BlockSpec((_GB, R, L), lambda b: (b, 0, 0)),
        ],
        out_specs=[
            pl.BlockSpec((_GB, L, R), lambda b: (b, 0, 0)),
            pl.BlockSpec((_GB, R, L), lambda b: (b, 0, 0)),
            pl.BlockSpec((_GB, R, L), lambda b: (b, 0, 0)),
        ],
        out_shape=[
            jax.ShapeDtypeStruct((B, L, R), jnp.int32),
            jax.ShapeDtypeStruct((B, R, L), jnp.float32),
            jax.ShapeDtypeStruct((B, R, L), jnp.float32),
        ],
        interpret=interpret,
    )(l0, l1)


_FCH = 32
_NCH = 8
_NBF = 4


def _gather_body(feats_hbm, ranks_hbm, l0_hbm, l1_hbm,
                 sf_hbm, p1_hbm, p0_hbm,
                 fidx, fbuf0, fbuf1, fbuf2, fbuf3, pidx, lbuf0, lbuf1,
                 stg0, stg1, gsem0, gsem1, gsem2, gsem3,
                 ssem0, ssem1, ssem2, ssem3):
    fbufs = (fbuf0, fbuf1, fbuf2, fbuf3)
    gsems = (gsem0, gsem1, gsem2, gsem3)
    ssems = (ssem0, ssem1, ssem2, ssem3)
    wid = lax.axis_index("s") * 2 + lax.axis_index("c")

    b = wid // 8
    row0 = pl.multiple_of(128 * b + 8 * ((wid % 8) // 2), 8)
    sub = 4 * ((wid % 8) % 2)
    pltpu.sync_copy(ranks_hbm.at[pl.ds(row0, 8)], fidx)

    def fire_gather(c):
        idx_ref = fidx.at[sub + c // 2, pl.ds(32 * (c % 2), 32)]
        return pltpu.async_copy(feats_hbm.at[idx_ref], fbufs[c % _NBF],
                                gsems[c % _NBF])

    def fire_store(c):
        out0 = pl.multiple_of(256 * wid + _FCH * c, _FCH)
        return pltpu.async_copy(fbufs[c % _NBF], sf_hbm.at[pl.ds(out0, _FCH)],
                                ssems[c % _NBF])

    gcp = [None] * _NBF
    scp = [None] * _NBF
    for c in range(_NBF):
        gcp[c] = fire_gather(c)
    for c in range(_NCH):
        if 1 <= c <= _NCH - _NBF:
            scp[(c - 1) % _NBF].wait()
            gcp[(c - 1) % _NBF] = fire_gather(c + _NBF - 1)
        gcp[c % _NBF].wait()
        scp[c % _NBF] = fire_store(c)
    for c in range(_NCH - _NBF, _NCH):
        scp[c % _NBF].wait()

    pltpu.sync_copy(ranks_hbm.at[pl.ds(pl.multiple_of(16 * wid, 8), 16)], pidx)
    pltpu.sync_copy(l0_hbm.at[pl.ds(pl.multiple_of(b * 64, 64), 64)], lbuf0)
    pltpu.sync_copy(l1_hbm.at[pl.ds(pl.multiple_of(b * 64, 64), 64)], lbuf1)
    base_flat = b * S
    for v in range(64):
        g = pidx[v // 4, pl.ds((v % 4) * 16, 16)]
        e = g - base_flat
        er, ec = e >> 7, e & 127
        g0 = plsc.load_gather(lbuf0, [er, ec])
        g1 = plsc.load_gather(lbuf1, [er, ec])
        stg0[v // 8, pl.ds((v % 8) * 16, 16)] = g0
        stg1[v // 8, pl.ds((v % 8) * 16, 16)] = g1
    jj0 = 1024 * (wid % 8)

    @pl.when(jj0 < K)
    def _():
        crow = pl.multiple_of(jj0 // 128, 8)
        pltpu.sync_copy(stg0, p1_hbm.at[2 * b, pl.ds(crow, 8)])
        pltpu.sync_copy(stg1, p1_hbm.at[2 * b + 1, pl.ds(crow, 8)])

    @pl.when(jj0 >= K)
    def _():
        crow = pl.multiple_of((jj0 - K) // 128, 8)
        pltpu.sync_copy(stg0, p0_hbm.at[2 * b, pl.ds(crow, 8)])
        pltpu.sync_copy(stg1, p0_hbm.at[2 * b + 1, pl.ds(crow, 8)])


@functools.lru_cache(maxsize=None)
def _build_gather():
    return pl.kernel(
        _gather_body,
        out_type=(
            jax.ShapeDtypeStruct((B * K, N), jnp.float32),
            jax.ShapeDtypeStruct((2 * B, K // 128, 128), jnp.float32),
            jax.ShapeDtypeStruct((2 * B, (S - K) // 128, 128), jnp.float32),
        ),
        mesh=plsc.VectorSubcoreMesh(core_axis_name="c", subcore_axis_name="s"),
        compiler_params=pltpu.CompilerParams(needs_layout_passes=False),
        scratch_types=[
            pltpu.VMEM((8, 64), jnp.int32),
            pltpu.VMEM((_FCH, N), jnp.float32),
            pltpu.VMEM((_FCH, N), jnp.float32),
            pltpu.VMEM((_FCH, N), jnp.float32),
            pltpu.VMEM((_FCH, N), jnp.float32),
            pltpu.VMEM((16, 64), jnp.int32),
            pltpu.VMEM((64, 128), jnp.float32),
            pltpu.VMEM((64, 128), jnp.float32),
            pltpu.VMEM((8, 128), jnp.float32),
            pltpu.VMEM((8, 128), jnp.float32),
            pltpu.SemaphoreType.DMA,
            pltpu.SemaphoreType.DMA,
            pltpu.SemaphoreType.DMA,
            pltpu.SemaphoreType.DMA,
            pltpu.SemaphoreType.DMA,
            pltpu.SemaphoreType.DMA,
            pltpu.SemaphoreType.DMA,
            pltpu.SemaphoreType.DMA,
        ],
    )


def _gather_call(feats2d, ranks256, l0p, l1p):
    return _build_gather()(feats2d, ranks256, l0p, l1p)


def kernel(feats, logit):
    l0 = logit[..., 0].reshape(B, R, L)
    l1 = logit[..., 1].reshape(B, R, L)
    ranks, l0s, l1s = _sort_call(l0, l1)
    ranks512 = ranks.reshape(B * S // 64, 64)
    feats2d = feats.reshape(B * S, N)
    l0p = l0s.reshape(B * S // 128, 128)
    l1p = l1s.reshape(B * S // 128, 128)
    sf2d, p1t, p0t = _gather_call(feats2d, ranks512, l0p, l1p)
    p1 = p1t.reshape(B, 2, K).swapaxes(1, 2)
    p0 = p0t.reshape(B, 2, S - K).swapaxes(1, 2)
    return (sf2d.reshape(B, K, N), p1, p0)


# speedup vs baseline: 1.0765x; 1.0180x over previous
import functools

import jax
import jax.numpy as jnp
from jax import lax
from jax.experimental import pallas as pl
from jax.experimental.pallas import tpu as pltpu
from jax.experimental.pallas import tpu_sc as plsc

B, S, K, N = 4, 8192, 2048, 768
R, L = 64, 128
NW = 32


_GB = 4


def _sort_body(l0_ref, l1_ref, ranks_ref, l0o_ref, l1o_ref):
    b = pl.program_id(0)
    l0 = l0_ref[...]
    l1 = l1_ref[...]
    l0o_ref[...] = l0
    l1o_ref[...] = l1
    m = jnp.maximum(l0, l1)
    e0 = jnp.exp(l0 - m)
    e1 = jnp.exp(l1 - m)
    key = jnp.maximum(e0, e1) / (e0 + e1)

    ri = lax.broadcasted_iota(jnp.int32, (_GB, R, L), 1)
    li = lax.broadcasted_iota(jnp.int32, (_GB, R, L), 2)
    bi = lax.broadcasted_iota(jnp.int32, (_GB, R, L), 0)
    gi = li * R + ri
    tok = ri * L + li
    idx = tok + (b * _GB + bi) * S

    def partner(x, mj, sh, ax):
        size = (_GB, R, L)[ax]
        return jnp.where(mj, pltpu.roll(x, sh, ax),
                         pltpu.roll(x, size - sh, ax))

    k = 2
    while k <= S:
        mk = (gi & k) != 0
        j = k // 2
        while j >= 1:
            mj = (gi & j) != 0
            ax, sh = (2, j // R) if j >= R else (1, j)
            pk = partner(key, mj, sh, ax)
            pi = partner(idx, mj, sh, ax)
            before = (pk > key) | ((pk == key) & (pi < idx))
            take = before ^ mj ^ mk
            key = jnp.where(take, pk, key)
            idx = jnp.where(take, pi, idx)
            j //= 2
        k *= 2
    ranks_ref[...] = jnp.swapaxes(idx, 1, 2)


def _sort_call(l0, l1, interpret=False):
    return pl.pallas_call(
        _sort_body,
        grid=(B // _GB,),
        in_specs=[
            pl.BlockSpec((_GB, R, L), lambda b: (b, 0, 0)),
            pl.BlockSpec((_GB, R, L), lambda b: (b, 0, 0)),
        ],
        out_specs=[
            pl.BlockSpec((_GB, L, R), lambda b: (b, 0, 0)),
            pl.BlockSpec((_GB, R, L), lambda b: (b, 0, 0)),
            pl.BlockSpec((_GB, R, L), lambda b: (b, 0, 0)),
        ],
        out_shape=[
            jax.ShapeDtypeStruct((B, L, R), jnp.int32),
            jax.ShapeDtypeStruct((B, R, L), jnp.float32),
            jax.ShapeDtypeStruct((B, R, L), jnp.float32),
        ],
        interpret=interpret,
    )(l0, l1)


_FCH = 32
_NCH = 8
_NBF = 4


def _gather_body(feats_hbm, ranks_hbm, l0_hbm, l1_hbm,
                 sf_hbm, p1_hbm, p0_hbm,
                 fidx, fbuf0, fbuf1, fbuf2, fbuf3, pidx, lbuf0, lbuf1,
                 stg0, stg1, gsem0, gsem1, gsem2, gsem3,
                 ssem0, ssem1, ssem2, ssem3):
    fbufs = (fbuf0, fbuf1, fbuf2, fbuf3)
    gsems = (gsem0, gsem1, gsem2, gsem3)
    ssems = (ssem0, ssem1, ssem2, ssem3)
    wid = lax.axis_index("s") * 2 + lax.axis_index("c")

    b = wid // 8
    row0 = pl.multiple_of(128 * b + 8 * ((wid % 8) // 2), 8)
    sub = 4 * ((wid % 8) % 2)
    pltpu.sync_copy(ranks_hbm.at[pl.ds(row0, 8)], fidx)

    def fire_gather(c):
        idx_ref = fidx.at[sub + c // 2, pl.ds(32 * (c % 2), 32)]
        return pltpu.async_copy(feats_hbm.at[idx_ref], fbufs[c % _NBF],
                                gsems[c % _NBF])

    def fire_store(c):
        out0 = pl.multiple_of(256 * wid + _FCH * c, _FCH)
        return pltpu.async_copy(fbufs[c % _NBF], sf_hbm.at[pl.ds(out0, _FCH)],
                                ssems[c % _NBF])

    gcp = [None] * _NBF
    scp = [None] * _NBF
    for c in range(_NBF):
        gcp[c] = fire_gather(c)

    pltpu.sync_copy(ranks_hbm.at[pl.ds(pl.multiple_of(16 * wid, 8), 16)], pidx)
    pltpu.sync_copy(l0_hbm.at[pl.ds(pl.multiple_of(b * 64, 64), 64)], lbuf0)
    pltpu.sync_copy(l1_hbm.at[pl.ds(pl.multiple_of(b * 64, 64), 64)], lbuf1)
    base_flat = b * S
    for v in range(64):
        g = pidx[v // 4, pl.ds((v % 4) * 16, 16)]
        e = g - base_flat
        er, ec = e >> 7, e & 127
        g0 = plsc.load_gather(lbuf0, [er, ec])
        g1 = plsc.load_gather(lbuf1, [er, ec])
        stg0[v // 8, pl.ds((v % 8) * 16, 16)] = g0
        stg1[v // 8, pl.ds((v % 8) * 16, 16)] = g1
    jj0 = 1024 * (wid % 8)

    @pl.when(jj0 < K)
    def _():
        crow = pl.multiple_of(jj0 // 128, 8)
        pltpu.sync_copy(stg0, p1_hbm.at[2 * b, pl.ds(crow, 8)])
        pltpu.sync_copy(stg1, p1_hbm.at[2 * b + 1, pl.ds(crow, 8)])

    @pl.when(jj0 >= K)
    def _():
        crow = pl.multiple_of((jj0 - K) // 128, 8)
        pltpu.sync_copy(stg0, p0_hbm.at[2 * b, pl.ds(crow, 8)])
        pltpu.sync_copy(stg1, p0_hbm.at[2 * b + 1, pl.ds(crow, 8)])
    for c in range(_NCH):
        if 1 <= c <= _NCH - _NBF:
            scp[(c - 1) % _NBF].wait()
            gcp[(c - 1) % _NBF] = fire_gather(c + _NBF - 1)
        gcp[c % _NBF].wait()
        scp[c % _NBF] = fire_store(c)
    for c in range(_NCH - _NBF, _NCH):
        scp[c % _NBF].wait()


@functools.lru_cache(maxsize=None)
def _build_gather():
    return pl.kernel(
        _gather_body,
        out_type=(
            jax.ShapeDtypeStruct((B * K, N), jnp.float32),
            jax.ShapeDtypeStruct((2 * B, K // 128, 128), jnp.float32),
            jax.ShapeDtypeStruct((2 * B, (S - K) // 128, 128), jnp.float32),
        ),
        mesh=plsc.VectorSubcoreMesh(core_axis_name="c", subcore_axis_name="s"),
        compiler_params=pltpu.CompilerParams(needs_layout_passes=False),
        scratch_types=[
            pltpu.VMEM((8, 64), jnp.int32),
            pltpu.VMEM((_FCH, N), jnp.float32),
            pltpu.VMEM((_FCH, N), jnp.float32),
            pltpu.VMEM((_FCH, N), jnp.float32),
            pltpu.VMEM((_FCH, N), jnp.float32),
            pltpu.VMEM((16, 64), jnp.int32),
            pltpu.VMEM((64, 128), jnp.float32),
            pltpu.VMEM((64, 128), jnp.float32),
            pltpu.VMEM((8, 128), jnp.float32),
            pltpu.VMEM((8, 128), jnp.float32),
            pltpu.SemaphoreType.DMA,
            pltpu.SemaphoreType.DMA,
            pltpu.SemaphoreType.DMA,
            pltpu.SemaphoreType.DMA,
            pltpu.SemaphoreType.DMA,
            pltpu.SemaphoreType.DMA,
            pltpu.SemaphoreType.DMA,
            pltpu.SemaphoreType.DMA,
        ],
    )


def _gather_call(feats2d, ranks256, l0p, l1p):
    return _build_gather()(feats2d, ranks256, l0p, l1p)


def kernel(feats, logit):
    l0 = logit[..., 0].reshape(B, R, L)
    l1 = logit[..., 1].reshape(B, R, L)
    ranks, l0s, l1s = _sort_call(l0, l1)
    ranks512 = ranks.reshape(B * S // 64, 64)
    feats2d = feats.reshape(B * S, N)
    l0p = l0s.reshape(B * S // 128, 128)
    l1p = l1s.reshape(B * S // 128, 128)
    sf2d, p1t, p0t = _gather_call(feats2d, ranks512, l0p, l1p)
    p1 = p1t.reshape(B, 2, K).swapaxes(1, 2)
    p0 = p0t.reshape(B, 2, S - K).swapaxes(1, 2)
    return (sf2d.reshape(B, K, N), p1, p0)
